# baseline (speedup 1.0000x reference)
import functools
import math

import jax
import jax.numpy as jnp
from jax import lax
from jax.experimental import pallas as pl
from jax.experimental.pallas import tpu as pltpu
from jax.experimental.pallas import tpu_sc as plsc

N_NODES = 100000
N_TYPES = 40
TEMPERATURE = 298.0
ION_STRENGTH = 0.05
CONSTANT = math.exp(-0.004314 * (TEMPERATURE - 273.0))
DIELEC = 8.8
IONIC_CORRECTED = 0.02 + ION_STRENGTH / 1.4
K_SCREEN = math.sqrt(200.0 * IONIC_CORRECTED / TEMPERATURE)
A_COEF = 332.0 / (DIELEC * CONSTANT)

NC = 2
NS = 16
NW = NC * NS
L = 16

ATOMS_PER_TILE = 3136
ATOMS_PAD = NW * ATOMS_PER_TILE

ROW_W = 8
ROWS_PER_TILE = 800
PAIRS_PAD = NW * ROWS_PER_TILE * 128
BLK_ROWS = 32
BLK_PAIRS = BLK_ROWS * 128
N_BLOCKS = ROWS_PER_TILE // BLK_ROWS


def _widx():
    return lax.axis_index("s") * NC + lax.axis_index("c")


def _iota16():
    return lax.iota(jnp.int32, L)


def _c16(v, dtype=jnp.int32):
    return jnp.full((L,), v, dtype)


def _sqrt16(x):
    i = lax.bitcast_convert_type(x, jnp.int32)
    i = jnp.int32(0x5F3759DF) - lax.shift_right_arithmetic(i, _c16(1))
    y = lax.bitcast_convert_type(i, jnp.float32)
    for _ in range(3):
        y = y * (1.5 - 0.5 * x * y * y)
    return x * y


def _build_table_kernel(coords_hbm, desc_hbm, props_hbm, table_hbm,
                        coords_v, desc_v, props_v, table_v):
    wid = _widx()
    base = wid * ATOMS_PER_TILE
    pltpu.sync_copy(coords_hbm.at[pl.ds(base, ATOMS_PER_TILE)], coords_v)
    pltpu.sync_copy(desc_hbm.at[pl.ds(base, ATOMS_PER_TILE)], desc_v)
    pltpu.sync_copy(props_hbm, props_v)

    def body(j, carry):
        rows = j * L + _iota16()
        atname = plsc.load_gather(desc_v, [rows, _c16(0)])
        resnum = plsc.load_gather(desc_v, [rows, _c16(1)])
        chain = plsc.load_gather(desc_v, [rows, _c16(2)])
        q = plsc.load_gather(props_v, [atname, _c16(1)])
        virt = plsc.load_gather(props_v, [atname, _c16(2)])
        qeff = jnp.where(virt == 1.0, 0.0, q)
        qcode = jnp.where(qeff > 0.0, 2, jnp.where(qeff < 0.0, 0, 1))
        code = qcode + 4 * (resnum * N_TYPES + chain)
        x = plsc.load_gather(coords_v, [rows, _c16(0)])
        y = plsc.load_gather(coords_v, [rows, _c16(1)])
        z = plsc.load_gather(coords_v, [rows, _c16(2)])
        plsc.store_scatter(table_v, [rows, _c16(0)], x)
        plsc.store_scatter(table_v, [rows, _c16(1)], y)
        plsc.store_scatter(table_v, [rows, _c16(2)], z)
        plsc.store_scatter(table_v, [rows, _c16(3)],
                           lax.bitcast_convert_type(code, jnp.float32))
        return carry

    lax.fori_loop(0, ATOMS_PER_TILE // L, body, 0, unroll=False)
    pltpu.sync_copy(table_v, table_hbm.at[pl.ds(base, ATOMS_PER_TILE)])


def _pair_energy_kernel(i0_hbm, i1_hbm, table_hbm, out_hbm,
                        idx0_a, idx1_a, rows0_a, rows1_a,
                        idx0_b, idx1_b, rows0_b, rows1_b,
                        acc_v, sem0_a, sem1_a, sem0_b, sem1_b):
    wid = _widx()
    seta = (idx0_a, idx1_a, rows0_a, rows1_a, sem0_a, sem1_a)
    setb = (idx0_b, idx1_b, rows0_b, rows1_b, sem0_b, sem1_b)

    def issue(bufs, g):
        idx0_v, idx1_v, rows0_v, rows1_v, s0, s1 = bufs
        row_off = wid * ROWS_PER_TILE + g * BLK_ROWS
        pltpu.sync_copy(i0_hbm.at[pl.ds(row_off, BLK_ROWS)], idx0_v)
        pltpu.sync_copy(i1_hbm.at[pl.ds(row_off, BLK_ROWS)], idx1_v)
        for k in range(BLK_ROWS):
            pltpu.async_copy(table_hbm.at[idx0_v.at[k]],
                             rows0_v.at[pl.ds(k * 128, 128)], s0)
            pltpu.async_copy(table_hbm.at[idx1_v.at[k]],
                             rows1_v.at[pl.ds(k * 128, 128)], s1)

    def drain(bufs):
        idx0_v, idx1_v, rows0_v, rows1_v, s0, s1 = bufs
        for k in range(BLK_ROWS):
            pltpu.make_async_copy(table_hbm.at[idx0_v.at[k]],
                                  rows0_v.at[pl.ds(k * 128, 128)], s0).wait()
            pltpu.make_async_copy(table_hbm.at[idx1_v.at[k]],
                                  rows1_v.at[pl.ds(k * 128, 128)], s1).wait()

    def compute(bufs, acc):
        _, _, rows0_v, rows1_v, _, _ = bufs

        def vreg(j, acc):
            rows = j * L + _iota16()
            x0 = plsc.load_gather(rows0_v, [rows, _c16(0)])
            y0 = plsc.load_gather(rows0_v, [rows, _c16(1)])
            z0 = plsc.load_gather(rows0_v, [rows, _c16(2)])
            c0 = lax.bitcast_convert_type(
                plsc.load_gather(rows0_v, [rows, _c16(3)]), jnp.int32)
            x1 = plsc.load_gather(rows1_v, [rows, _c16(0)])
            y1 = plsc.load_gather(rows1_v, [rows, _c16(1)])
            z1 = plsc.load_gather(rows1_v, [rows, _c16(2)])
            c1 = lax.bitcast_convert_type(
                plsc.load_gather(rows1_v, [rows, _c16(3)]), jnp.int32)
            qc0 = c0 & 3
            qc1 = c1 & 3
            meta0 = lax.shift_right_arithmetic(c0, _c16(2))
            meta1 = lax.shift_right_arithmetic(c1, _c16(2))
            dx = x0 - x1
            dy = y0 - y1
            dz = z0 - z1
            d2 = dx * dx + dy * dy + dz * dz
            valid = ((qc0 != 1) & (qc1 != 1) & (meta0 != meta1)
                     & (d2 <= 225.0))
            s = ((qc0 - 1) * (qc1 - 1)).astype(jnp.float32)
            dist = _sqrt16(d2 + 1e-12)
            d28 = jnp.maximum(dist, 2.8)
            e = s * (A_COEF * jnp.exp(-K_SCREEN * d28)) / (d28 * d28)
            return acc + jnp.where(valid, e, 0.0)

        return lax.fori_loop(0, BLK_PAIRS // L, vreg, acc, unroll=False)

    def step(g, acc):
        issue(seta, g)
        drain(seta)
        return compute(seta, acc)

    acc = lax.fori_loop(0, N_BLOCKS, step,
                        jnp.zeros((L,), jnp.float32), unroll=False)
    del setb
    acc_v[...] = acc
    pltpu.sync_copy(acc_v, out_hbm.at[wid])


def kernel(coords, partners, atom_description, atomPairs, hbondNet,
           alternativeMask, weight, atom_Properties,
           calculate_helical_dipoles=0):
    del partners, hbondNet, alternativeMask, calculate_helical_dipoles
    mesh = plsc.VectorSubcoreMesh(core_axis_name="c", subcore_axis_name="s")

    coords_p = jnp.zeros((ATOMS_PAD, 3), jnp.float32).at[:N_NODES].set(coords)
    desc_p = jnp.zeros((ATOMS_PAD, 3), jnp.int32).at[:N_NODES].set(
        atom_description)

    build = functools.partial(
        pl.kernel,
        out_type=jax.ShapeDtypeStruct((ATOMS_PAD, ROW_W), jnp.float32),
        mesh=mesh,
        compiler_params=pltpu.CompilerParams(needs_layout_passes=False, use_tc_tiling_on_sc=False),
        scratch_types=[
            pltpu.VMEM((ATOMS_PER_TILE, 3), jnp.float32),
            pltpu.VMEM((ATOMS_PER_TILE, 3), jnp.int32),
            pltpu.VMEM((N_TYPES, 3), jnp.float32),
            pltpu.VMEM((ATOMS_PER_TILE, ROW_W), jnp.float32),
        ],
    )(_build_table_kernel)
    table = build(coords_p, desc_p, atom_Properties)

    n_pairs = atomPairs.shape[0]
    i0 = jnp.zeros((PAIRS_PAD,), jnp.int32).at[:n_pairs].set(atomPairs[:, 0])
    i1 = jnp.zeros((PAIRS_PAD,), jnp.int32).at[:n_pairs].set(atomPairs[:, 1])
    i0 = i0.reshape(PAIRS_PAD // 128, 128)
    i1 = i1.reshape(PAIRS_PAD // 128, 128)

    energy = functools.partial(
        pl.kernel,
        out_type=jax.ShapeDtypeStruct((NW, L), jnp.float32),
        mesh=mesh,
        compiler_params=pltpu.CompilerParams(needs_layout_passes=False, use_tc_tiling_on_sc=False),
        scratch_types=[
            pltpu.VMEM((BLK_ROWS, 128), jnp.int32),
            pltpu.VMEM((BLK_ROWS, 128), jnp.int32),
            pltpu.VMEM((BLK_PAIRS, ROW_W), jnp.float32),
            pltpu.VMEM((BLK_PAIRS, ROW_W), jnp.float32),
            pltpu.VMEM((BLK_ROWS, 128), jnp.int32),
            pltpu.VMEM((BLK_ROWS, 128), jnp.int32),
            pltpu.VMEM((BLK_PAIRS, ROW_W), jnp.float32),
            pltpu.VMEM((BLK_PAIRS, ROW_W), jnp.float32),
            pltpu.VMEM((L,), jnp.float32),
            pltpu.SemaphoreType.DMA,
            pltpu.SemaphoreType.DMA,
            pltpu.SemaphoreType.DMA,
            pltpu.SemaphoreType.DMA,
        ],
    )(_pair_energy_kernel)
    partials = energy(i0, i1, table)

    return jnp.sum(partials) * (1.0 + weight[0])


# speedup vs baseline: 1.0002x; 1.0002x over previous
import functools
import math

import jax
import jax.numpy as jnp
from jax import lax
from jax.experimental import pallas as pl
from jax.experimental.pallas import tpu as pltpu
from jax.experimental.pallas import tpu_sc as plsc

N_NODES = 100000
N_TYPES = 40
TEMPERATURE = 298.0
ION_STRENGTH = 0.05
CONSTANT = math.exp(-0.004314 * (TEMPERATURE - 273.0))
DIELEC = 8.8
IONIC_CORRECTED = 0.02 + ION_STRENGTH / 1.4
K_SCREEN = math.sqrt(200.0 * IONIC_CORRECTED / TEMPERATURE)
A_COEF = 332.0 / (DIELEC * CONSTANT)

NC = 2
NS = 16
NW = NC * NS
L = 16

ATOMS_PER_TILE = 3136
ATOMS_PAD = NW * ATOMS_PER_TILE

ROW_W = 8
ROWS_PER_TILE = 800
PAIRS_PAD = NW * ROWS_PER_TILE * 128
BLK_ROWS = 16
BLK_PAIRS = BLK_ROWS * 128
N_BLOCKS = ROWS_PER_TILE // BLK_ROWS


def _widx():
    return lax.axis_index("s") * NC + lax.axis_index("c")


def _iota16():
    return lax.iota(jnp.int32, L)


def _c16(v, dtype=jnp.int32):
    return jnp.full((L,), v, dtype)


def _sqrt16(x):
    i = lax.bitcast_convert_type(x, jnp.int32)
    i = jnp.int32(0x5F3759DF) - lax.shift_right_arithmetic(i, _c16(1))
    y = lax.bitcast_convert_type(i, jnp.float32)
    for _ in range(3):
        y = y * (1.5 - 0.5 * x * y * y)
    return x * y


def _build_table_kernel(coords_hbm, desc_hbm, props_hbm, table_hbm,
                        coords_v, desc_v, props_v, table_v):
    wid = _widx()
    base = wid * ATOMS_PER_TILE
    pltpu.sync_copy(coords_hbm.at[pl.ds(base, ATOMS_PER_TILE)], coords_v)
    pltpu.sync_copy(desc_hbm.at[pl.ds(base, ATOMS_PER_TILE)], desc_v)
    pltpu.sync_copy(props_hbm, props_v)

    def body(j, carry):
        rows = j * L + _iota16()
        atname = plsc.load_gather(desc_v, [rows, _c16(0)])
        resnum = plsc.load_gather(desc_v, [rows, _c16(1)])
        chain = plsc.load_gather(desc_v, [rows, _c16(2)])
        q = plsc.load_gather(props_v, [atname, _c16(1)])
        virt = plsc.load_gather(props_v, [atname, _c16(2)])
        qeff = jnp.where(virt == 1.0, 0.0, q)
        qcode = jnp.where(qeff > 0.0, 2, jnp.where(qeff < 0.0, 0, 1))
        code = qcode + 4 * (resnum * N_TYPES + chain)
        x = plsc.load_gather(coords_v, [rows, _c16(0)])
        y = plsc.load_gather(coords_v, [rows, _c16(1)])
        z = plsc.load_gather(coords_v, [rows, _c16(2)])
        plsc.store_scatter(table_v, [rows, _c16(0)], x)
        plsc.store_scatter(table_v, [rows, _c16(1)], y)
        plsc.store_scatter(table_v, [rows, _c16(2)], z)
        plsc.store_scatter(table_v, [rows, _c16(3)],
                           lax.bitcast_convert_type(code, jnp.float32))
        return carry

    lax.fori_loop(0, ATOMS_PER_TILE // L, body, 0, unroll=False)
    pltpu.sync_copy(table_v, table_hbm.at[pl.ds(base, ATOMS_PER_TILE)])


def _pair_energy_kernel(i0_hbm, i1_hbm, table_hbm, out_hbm,
                        idx0_a, idx1_a, rows0_a, rows1_a,
                        acc_v, sem0_a, sem1_a):
    wid = _widx()
    seta = (idx0_a, idx1_a, rows0_a, rows1_a, sem0_a, sem1_a)

    def issue(bufs, g):
        idx0_v, idx1_v, rows0_v, rows1_v, s0, s1 = bufs
        row_off = wid * ROWS_PER_TILE + g * BLK_ROWS
        pltpu.sync_copy(i0_hbm.at[pl.ds(row_off, BLK_ROWS)], idx0_v)
        pltpu.sync_copy(i1_hbm.at[pl.ds(row_off, BLK_ROWS)], idx1_v)
        for k in range(BLK_ROWS):
            pltpu.async_copy(table_hbm.at[idx0_v.at[k]],
                             rows0_v.at[pl.ds(k * 128, 128)], s0)
            pltpu.async_copy(table_hbm.at[idx1_v.at[k]],
                             rows1_v.at[pl.ds(k * 128, 128)], s1)

    def drain(bufs):
        idx0_v, idx1_v, rows0_v, rows1_v, s0, s1 = bufs
        for k in range(BLK_ROWS):
            pltpu.make_async_copy(table_hbm.at[idx0_v.at[k]],
                                  rows0_v.at[pl.ds(k * 128, 128)], s0).wait()
            pltpu.make_async_copy(table_hbm.at[idx1_v.at[k]],
                                  rows1_v.at[pl.ds(k * 128, 128)], s1).wait()

    def compute(bufs, acc):
        _, _, rows0_v, rows1_v, _, _ = bufs

        def vreg(j, acc):
            rows = j * L + _iota16()
            x0 = plsc.load_gather(rows0_v, [rows, _c16(0)])
            y0 = plsc.load_gather(rows0_v, [rows, _c16(1)])
            z0 = plsc.load_gather(rows0_v, [rows, _c16(2)])
            c0 = lax.bitcast_convert_type(
                plsc.load_gather(rows0_v, [rows, _c16(3)]), jnp.int32)
            x1 = plsc.load_gather(rows1_v, [rows, _c16(0)])
            y1 = plsc.load_gather(rows1_v, [rows, _c16(1)])
            z1 = plsc.load_gather(rows1_v, [rows, _c16(2)])
            c1 = lax.bitcast_convert_type(
                plsc.load_gather(rows1_v, [rows, _c16(3)]), jnp.int32)
            qc0 = c0 & 3
            qc1 = c1 & 3
            meta0 = lax.shift_right_arithmetic(c0, _c16(2))
            meta1 = lax.shift_right_arithmetic(c1, _c16(2))
            dx = x0 - x1
            dy = y0 - y1
            dz = z0 - z1
            d2 = dx * dx + dy * dy + dz * dz
            valid = ((qc0 != 1) & (qc1 != 1) & (meta0 != meta1)
                     & (d2 <= 225.0))
            s = ((qc0 - 1) * (qc1 - 1)).astype(jnp.float32)
            dist = _sqrt16(d2 + 1e-12)
            d28 = jnp.maximum(dist, 2.8)
            e = s * (A_COEF * jnp.exp(-K_SCREEN * d28)) / (d28 * d28)
            return acc + jnp.where(valid, e, 0.0)

        return lax.fori_loop(0, BLK_PAIRS // L, vreg, acc, unroll=False)

    def step(g, acc):
        issue(seta, g)
        drain(seta)
        return compute(seta, acc)

    acc = lax.fori_loop(0, N_BLOCKS, step,
                        jnp.zeros((L,), jnp.float32), unroll=False)
    acc_v[...] = acc
    pltpu.sync_copy(acc_v, out_hbm.at[wid])


def kernel(coords, partners, atom_description, atomPairs, hbondNet,
           alternativeMask, weight, atom_Properties,
           calculate_helical_dipoles=0):
    del partners, hbondNet, alternativeMask, calculate_helical_dipoles
    mesh = plsc.VectorSubcoreMesh(core_axis_name="c", subcore_axis_name="s")

    coords_p = jnp.zeros((ATOMS_PAD, 3), jnp.float32).at[:N_NODES].set(coords)
    desc_p = jnp.zeros((ATOMS_PAD, 3), jnp.int32).at[:N_NODES].set(
        atom_description)

    build = functools.partial(
        pl.kernel,
        out_type=jax.ShapeDtypeStruct((ATOMS_PAD, ROW_W), jnp.float32),
        mesh=mesh,
        compiler_params=pltpu.CompilerParams(needs_layout_passes=False, use_tc_tiling_on_sc=False),
        scratch_types=[
            pltpu.VMEM((ATOMS_PER_TILE, 3), jnp.float32),
            pltpu.VMEM((ATOMS_PER_TILE, 3), jnp.int32),
            pltpu.VMEM((N_TYPES, 3), jnp.float32),
            pltpu.VMEM((ATOMS_PER_TILE, ROW_W), jnp.float32),
        ],
    )(_build_table_kernel)
    table = build(coords_p, desc_p, atom_Properties)

    n_pairs = atomPairs.shape[0]
    i0 = jnp.zeros((PAIRS_PAD,), jnp.int32).at[:n_pairs].set(atomPairs[:, 0])
    i1 = jnp.zeros((PAIRS_PAD,), jnp.int32).at[:n_pairs].set(atomPairs[:, 1])
    i0 = i0.reshape(PAIRS_PAD // 128, 128)
    i1 = i1.reshape(PAIRS_PAD // 128, 128)

    energy = functools.partial(
        pl.kernel,
        out_type=jax.ShapeDtypeStruct((NW, L), jnp.float32),
        mesh=mesh,
        compiler_params=pltpu.CompilerParams(needs_layout_passes=False, use_tc_tiling_on_sc=False),
        scratch_types=[
            pltpu.VMEM((BLK_ROWS, 128), jnp.int32),
            pltpu.VMEM((BLK_ROWS, 128), jnp.int32),
            pltpu.VMEM((BLK_PAIRS, ROW_W), jnp.float32),
            pltpu.VMEM((BLK_PAIRS, ROW_W), jnp.float32),
            pltpu.VMEM((L,), jnp.float32),
            pltpu.SemaphoreType.DMA,
            pltpu.SemaphoreType.DMA,
        ],
    )(_pair_energy_kernel)
    partials = energy(i0, i1, table)

    return jnp.sum(partials) * (1.0 + weight[0])


# speedup vs baseline: 1.8491x; 1.8487x over previous
import functools
import math

import jax
import jax.numpy as jnp
from jax import lax
from jax.experimental import pallas as pl
from jax.experimental.pallas import tpu as pltpu
from jax.experimental.pallas import tpu_sc as plsc

N_NODES = 100000
N_TYPES = 40
TEMPERATURE = 298.0
ION_STRENGTH = 0.05
CONSTANT = math.exp(-0.004314 * (TEMPERATURE - 273.0))
DIELEC = 8.8
IONIC_CORRECTED = 0.02 + ION_STRENGTH / 1.4
K_SCREEN = math.sqrt(200.0 * IONIC_CORRECTED / TEMPERATURE)
A_COEF = 332.0 / (DIELEC * CONSTANT)

NC = 2
NS = 16
NW = NC * NS
L = 16

ATOMS_PER_TILE = 3136
ATOMS_PAD = NW * ATOMS_PER_TILE

ROW_W = 8
ROWS_PER_TILE = 784
PAIRS_PAD = NW * ROWS_PER_TILE * 128
BLK_ROWS = 16
BLK_PAIRS = BLK_ROWS * 128
N_BLOCKS = ROWS_PER_TILE // BLK_ROWS


def _widx():
    return lax.axis_index("s") * NC + lax.axis_index("c")


def _iota16():
    return lax.iota(jnp.int32, L)


def _c16(v, dtype=jnp.int32):
    return jnp.full((L,), v, dtype)


def _sqrt16(x):
    i = lax.bitcast_convert_type(x, jnp.int32)
    i = jnp.int32(0x5F3759DF) - lax.shift_right_arithmetic(i, _c16(1))
    y = lax.bitcast_convert_type(i, jnp.float32)
    for _ in range(3):
        y = y * (1.5 - 0.5 * x * y * y)
    return x * y


def _build_table_kernel(coords_hbm, desc_hbm, props_hbm, table_hbm,
                        coords_v, desc_v, props_v, table_v):
    wid = _widx()
    base = wid * ATOMS_PER_TILE
    pltpu.sync_copy(coords_hbm.at[pl.ds(base, ATOMS_PER_TILE)], coords_v)
    pltpu.sync_copy(desc_hbm.at[pl.ds(base, ATOMS_PER_TILE)], desc_v)
    pltpu.sync_copy(props_hbm, props_v)

    def body(j, carry):
        rows = j * L + _iota16()
        atname = plsc.load_gather(desc_v, [rows, _c16(0)])
        resnum = plsc.load_gather(desc_v, [rows, _c16(1)])
        chain = plsc.load_gather(desc_v, [rows, _c16(2)])
        q = plsc.load_gather(props_v, [atname, _c16(1)])
        virt = plsc.load_gather(props_v, [atname, _c16(2)])
        qeff = jnp.where(virt == 1.0, 0.0, q)
        qcode = jnp.where(qeff > 0.0, 2, jnp.where(qeff < 0.0, 0, 1))
        code = qcode + 4 * (resnum * N_TYPES + chain)
        x = plsc.load_gather(coords_v, [rows, _c16(0)])
        y = plsc.load_gather(coords_v, [rows, _c16(1)])
        z = plsc.load_gather(coords_v, [rows, _c16(2)])
        plsc.store_scatter(table_v, [rows, _c16(0)], x)
        plsc.store_scatter(table_v, [rows, _c16(1)], y)
        plsc.store_scatter(table_v, [rows, _c16(2)], z)
        plsc.store_scatter(table_v, [rows, _c16(3)],
                           lax.bitcast_convert_type(code, jnp.float32))
        return carry

    lax.fori_loop(0, ATOMS_PER_TILE // L, body, 0, unroll=False)
    pltpu.sync_copy(table_v, table_hbm.at[pl.ds(base, ATOMS_PER_TILE)])


def _pair_energy_kernel(i0_hbm, i1_hbm, table_hbm, out_hbm,
                        idx0_a, idx1_a, rows0_a, rows1_a,
                        acc_v, sem0_a, sem1_a):
    wid = _widx()
    seta = (idx0_a, idx1_a, rows0_a, rows1_a, sem0_a, sem1_a)

    def issue(bufs, g):
        idx0_v, idx1_v, rows0_v, rows1_v, s0, s1 = bufs
        row_off = wid * ROWS_PER_TILE + g * BLK_ROWS
        pltpu.sync_copy(i0_hbm.at[pl.ds(row_off, BLK_ROWS)], idx0_v)
        pltpu.sync_copy(i1_hbm.at[pl.ds(row_off, BLK_ROWS)], idx1_v)
        for k in range(BLK_ROWS):
            pltpu.async_copy(table_hbm.at[idx0_v.at[k]],
                             rows0_v.at[pl.ds(k * 128, 128)], s0)
            pltpu.async_copy(table_hbm.at[idx1_v.at[k]],
                             rows1_v.at[pl.ds(k * 128, 128)], s1)

    def drain(bufs):
        idx0_v, idx1_v, rows0_v, rows1_v, s0, s1 = bufs
        for k in range(BLK_ROWS):
            pltpu.make_async_copy(table_hbm.at[idx0_v.at[k]],
                                  rows0_v.at[pl.ds(k * 128, 128)], s0).wait()
            pltpu.make_async_copy(table_hbm.at[idx1_v.at[k]],
                                  rows1_v.at[pl.ds(k * 128, 128)], s1).wait()

    def compute(bufs, acc):
        _, _, rows0_v, rows1_v, _, _ = bufs

        def vreg(j, acc):
            rows = j * L + _iota16()
            x0 = plsc.load_gather(rows0_v, [rows, _c16(0)])
            y0 = plsc.load_gather(rows0_v, [rows, _c16(1)])
            z0 = plsc.load_gather(rows0_v, [rows, _c16(2)])
            c0 = lax.bitcast_convert_type(
                plsc.load_gather(rows0_v, [rows, _c16(3)]), jnp.int32)
            x1 = plsc.load_gather(rows1_v, [rows, _c16(0)])
            y1 = plsc.load_gather(rows1_v, [rows, _c16(1)])
            z1 = plsc.load_gather(rows1_v, [rows, _c16(2)])
            c1 = lax.bitcast_convert_type(
                plsc.load_gather(rows1_v, [rows, _c16(3)]), jnp.int32)
            qc0 = c0 & 3
            qc1 = c1 & 3
            meta0 = lax.shift_right_arithmetic(c0, _c16(2))
            meta1 = lax.shift_right_arithmetic(c1, _c16(2))
            dx = x0 - x1
            dy = y0 - y1
            dz = z0 - z1
            d2 = dx * dx + dy * dy + dz * dz
            valid = ((qc0 != 1) & (qc1 != 1) & (meta0 != meta1)
                     & (d2 <= 225.0))
            s = ((qc0 - 1) * (qc1 - 1)).astype(jnp.float32)
            dist = _sqrt16(d2 + 1e-12)
            d28 = jnp.maximum(dist, 2.8)
            e = s * (A_COEF * jnp.exp(-K_SCREEN * d28)) / (d28 * d28)
            return acc + jnp.where(valid, e, 0.0)

        return lax.fori_loop(0, BLK_PAIRS // L, vreg, acc, unroll=False)

    def step(g, acc):
        issue(seta, g)
        drain(seta)
        return compute(seta, acc)

    acc = lax.fori_loop(0, N_BLOCKS, step,
                        jnp.zeros((L,), jnp.float32), unroll=False)
    acc_v[...] = acc
    pltpu.sync_copy(acc_v, out_hbm.at[wid])


def kernel(coords, partners, atom_description, atomPairs, hbondNet,
           alternativeMask, weight, atom_Properties,
           calculate_helical_dipoles=0):
    del partners, hbondNet, alternativeMask, calculate_helical_dipoles
    mesh = plsc.VectorSubcoreMesh(core_axis_name="c", subcore_axis_name="s")

    coords_p = jnp.zeros((ATOMS_PAD, 3), jnp.float32).at[:N_NODES].set(coords)
    desc_p = jnp.zeros((ATOMS_PAD, 3), jnp.int32).at[:N_NODES].set(
        atom_description)

    build = functools.partial(
        pl.kernel,
        out_type=jax.ShapeDtypeStruct((ATOMS_PAD, ROW_W), jnp.float32),
        mesh=mesh,
        compiler_params=pltpu.CompilerParams(needs_layout_passes=False, use_tc_tiling_on_sc=False),
        scratch_types=[
            pltpu.VMEM((ATOMS_PER_TILE, 3), jnp.float32),
            pltpu.VMEM((ATOMS_PER_TILE, 3), jnp.int32),
            pltpu.VMEM((N_TYPES, 3), jnp.float32),
            pltpu.VMEM((ATOMS_PER_TILE, ROW_W), jnp.float32),
        ],
    )(_build_table_kernel)
    table = build(coords_p, desc_p, atom_Properties)

    n_pairs = atomPairs.shape[0]
    i0 = jnp.zeros((PAIRS_PAD,), jnp.int32).at[:n_pairs].set(atomPairs[:, 0])
    i1 = jnp.zeros((PAIRS_PAD,), jnp.int32).at[:n_pairs].set(atomPairs[:, 1])
    i0 = i0.reshape(PAIRS_PAD // 128, 128)
    i1 = i1.reshape(PAIRS_PAD // 128, 128)

    energy = functools.partial(
        pl.kernel,
        out_type=jax.ShapeDtypeStruct((NW, L), jnp.float32),
        mesh=mesh,
        compiler_params=pltpu.CompilerParams(needs_layout_passes=False, use_tc_tiling_on_sc=False),
        scratch_types=[
            pltpu.VMEM((BLK_ROWS, 128), jnp.int32),
            pltpu.VMEM((BLK_ROWS, 128), jnp.int32),
            pltpu.VMEM((BLK_PAIRS, ROW_W), jnp.float32),
            pltpu.VMEM((BLK_PAIRS, ROW_W), jnp.float32),
            pltpu.VMEM((L,), jnp.float32),
            pltpu.SemaphoreType.DMA,
            pltpu.SemaphoreType.DMA,
        ],
    )(_pair_energy_kernel)
    partials = energy(i0, i1, table)

    return jnp.sum(partials) * (1.0 + weight[0])


# speedup vs baseline: 2.0635x; 1.1159x over previous
import functools
import math

import jax
import jax.numpy as jnp
from jax import lax
from jax.experimental import pallas as pl
from jax.experimental.pallas import tpu as pltpu
from jax.experimental.pallas import tpu_sc as plsc

N_NODES = 100000
N_TYPES = 40
TEMPERATURE = 298.0
ION_STRENGTH = 0.05
CONSTANT = math.exp(-0.004314 * (TEMPERATURE - 273.0))
DIELEC = 8.8
IONIC_CORRECTED = 0.02 + ION_STRENGTH / 1.4
K_SCREEN = math.sqrt(200.0 * IONIC_CORRECTED / TEMPERATURE)
A_COEF = 332.0 / (DIELEC * CONSTANT)

NC = 2
NS = 16
NW = NC * NS
L = 16

ATOMS_PER_TILE = 3136
ATOMS_PAD = NW * ATOMS_PER_TILE

ROW_W = 8
ROWS_PER_TILE = 784
PAIRS_PAD = NW * ROWS_PER_TILE * 128
BLK_ROWS = 16
BLK_PAIRS = BLK_ROWS * 128
N_BLOCKS = ROWS_PER_TILE // BLK_ROWS
IDX_ROWS = PAIRS_PAD // 128


def _widx():
    return lax.axis_index("s") * NC + lax.axis_index("c")


def _iota16():
    return lax.iota(jnp.int32, L)


def _c16(v, dtype=jnp.int32):
    return jnp.full((L,), v, dtype)


def _sqrt16(x):
    i = lax.bitcast_convert_type(x, jnp.int32)
    i = jnp.int32(0x5F3759DF) - lax.shift_right_arithmetic(i, _c16(1))
    y = lax.bitcast_convert_type(i, jnp.float32)
    for _ in range(3):
        y = y * (1.5 - 0.5 * x * y * y)
    return x * y


def _build_table_kernel(coords_hbm, desc_hbm, props_hbm, table_hbm,
                        coords_v, desc_v, props_v, table_v):
    wid = _widx()
    base = wid * ATOMS_PER_TILE
    pltpu.sync_copy(coords_hbm.at[pl.ds(base, ATOMS_PER_TILE)], coords_v)
    pltpu.sync_copy(desc_hbm.at[pl.ds(base, ATOMS_PER_TILE)], desc_v)
    pltpu.sync_copy(props_hbm, props_v)

    def body(j, carry):
        rows = j * L + _iota16()
        atname = plsc.load_gather(desc_v, [rows, _c16(0)])
        resnum = plsc.load_gather(desc_v, [rows, _c16(1)])
        chain = plsc.load_gather(desc_v, [rows, _c16(2)])
        q = plsc.load_gather(props_v, [atname, _c16(1)])
        virt = plsc.load_gather(props_v, [atname, _c16(2)])
        qeff = jnp.where(virt == 1.0, 0.0, q)
        qcode = jnp.where(qeff > 0.0, 2, jnp.where(qeff < 0.0, 0, 1))
        code = qcode + 4 * (resnum * N_TYPES + chain)
        x = plsc.load_gather(coords_v, [rows, _c16(0)])
        y = plsc.load_gather(coords_v, [rows, _c16(1)])
        z = plsc.load_gather(coords_v, [rows, _c16(2)])
        plsc.store_scatter(table_v, [rows, _c16(0)], x)
        plsc.store_scatter(table_v, [rows, _c16(1)], y)
        plsc.store_scatter(table_v, [rows, _c16(2)], z)
        plsc.store_scatter(table_v, [rows, _c16(3)],
                           lax.bitcast_convert_type(code, jnp.float32))
        return carry

    lax.fori_loop(0, ATOMS_PER_TILE // L, body, 0, unroll=False)
    pltpu.sync_copy(table_v, table_hbm.at[pl.ds(base, ATOMS_PER_TILE)])


def _pair_energy_kernel(i0_hbm, i1_hbm, table_hbm, out_hbm,
                        idx0_a, idx1_a, rows0_a, rows1_a,
                        idx0_b, idx1_b, rows0_b, rows1_b,
                        acc_v, sem0_a, sem1_a, sem0_b, sem1_b):
    wid = _widx()
    seta = (idx0_a, idx1_a, rows0_a, rows1_a, sem0_a, sem1_a)
    setb = (idx0_b, idx1_b, rows0_b, rows1_b, sem0_b, sem1_b)

    def issue(bufs, g):
        idx0_v, idx1_v, rows0_v, rows1_v, s0, s1 = bufs
        row_off = wid * ROWS_PER_TILE + g * BLK_ROWS
        pltpu.sync_copy(i0_hbm.at[pl.ds(row_off, BLK_ROWS)], idx0_v)
        pltpu.sync_copy(i1_hbm.at[pl.ds(row_off, BLK_ROWS)], idx1_v)
        for k in range(BLK_ROWS):
            pltpu.async_copy(table_hbm.at[idx0_v.at[k]],
                             rows0_v.at[pl.ds(k * 128, 128)], s0)
            pltpu.async_copy(table_hbm.at[idx1_v.at[k]],
                             rows1_v.at[pl.ds(k * 128, 128)], s1)

    def drain(bufs):
        idx0_v, idx1_v, rows0_v, rows1_v, s0, s1 = bufs
        for k in range(BLK_ROWS):
            pltpu.make_async_copy(table_hbm.at[idx0_v.at[k]],
                                  rows0_v.at[pl.ds(k * 128, 128)], s0).wait()
            pltpu.make_async_copy(table_hbm.at[idx1_v.at[k]],
                                  rows1_v.at[pl.ds(k * 128, 128)], s1).wait()

    def compute(bufs, acc):
        _, _, rows0_v, rows1_v, _, _ = bufs

        def vreg(j, acc):
            rows = j * L + _iota16()
            x0 = plsc.load_gather(rows0_v, [rows, _c16(0)])
            y0 = plsc.load_gather(rows0_v, [rows, _c16(1)])
            z0 = plsc.load_gather(rows0_v, [rows, _c16(2)])
            c0 = lax.bitcast_convert_type(
                plsc.load_gather(rows0_v, [rows, _c16(3)]), jnp.int32)
            x1 = plsc.load_gather(rows1_v, [rows, _c16(0)])
            y1 = plsc.load_gather(rows1_v, [rows, _c16(1)])
            z1 = plsc.load_gather(rows1_v, [rows, _c16(2)])
            c1 = lax.bitcast_convert_type(
                plsc.load_gather(rows1_v, [rows, _c16(3)]), jnp.int32)
            qc0 = c0 & 3
            qc1 = c1 & 3
            meta0 = lax.shift_right_arithmetic(c0, _c16(2))
            meta1 = lax.shift_right_arithmetic(c1, _c16(2))
            dx = x0 - x1
            dy = y0 - y1
            dz = z0 - z1
            d2 = dx * dx + dy * dy + dz * dz
            valid = ((qc0 != 1) & (qc1 != 1) & (meta0 != meta1)
                     & (d2 <= 225.0))
            s = ((qc0 - 1) * (qc1 - 1)).astype(jnp.float32)
            dist = _sqrt16(d2 + 1e-12)
            d28 = jnp.maximum(dist, 2.8)
            e = s * (A_COEF * jnp.exp(-K_SCREEN * d28)) / (d28 * d28)
            return acc + jnp.where(valid, e, 0.0)

        return lax.fori_loop(0, BLK_PAIRS // L, vreg, acc, unroll=False)

    def step(g, acc):
        drain(seta)
        issue(setb, 2 * g + 1)
        acc = compute(seta, acc)
        drain(setb)
        issue(seta, 2 * g + 2)
        return compute(setb, acc)

    issue(seta, 0)
    acc = lax.fori_loop(0, N_BLOCKS // 2, step,
                        jnp.zeros((L,), jnp.float32), unroll=False)
    drain(seta)
    acc = compute(seta, acc)
    acc_v[...] = acc
    pltpu.sync_copy(acc_v, out_hbm.at[wid])


def kernel(coords, partners, atom_description, atomPairs, hbondNet,
           alternativeMask, weight, atom_Properties,
           calculate_helical_dipoles=0):
    del partners, hbondNet, alternativeMask, calculate_helical_dipoles
    mesh = plsc.VectorSubcoreMesh(core_axis_name="c", subcore_axis_name="s")

    coords_p = jnp.zeros((ATOMS_PAD, 3), jnp.float32).at[:N_NODES].set(coords)
    desc_p = jnp.zeros((ATOMS_PAD, 3), jnp.int32).at[:N_NODES].set(
        atom_description)

    build = functools.partial(
        pl.kernel,
        out_type=jax.ShapeDtypeStruct((ATOMS_PAD, ROW_W), jnp.float32),
        mesh=mesh,
        compiler_params=pltpu.CompilerParams(needs_layout_passes=False, use_tc_tiling_on_sc=False),
        scratch_types=[
            pltpu.VMEM((ATOMS_PER_TILE, 3), jnp.float32),
            pltpu.VMEM((ATOMS_PER_TILE, 3), jnp.int32),
            pltpu.VMEM((N_TYPES, 3), jnp.float32),
            pltpu.VMEM((ATOMS_PER_TILE, ROW_W), jnp.float32),
        ],
    )(_build_table_kernel)
    table = build(coords_p, desc_p, atom_Properties)

    n_pairs = atomPairs.shape[0]
    i0 = jnp.zeros((IDX_ROWS * 128,), jnp.int32).at[:n_pairs].set(
        atomPairs[:, 0])
    i1 = jnp.zeros((IDX_ROWS * 128,), jnp.int32).at[:n_pairs].set(
        atomPairs[:, 1])
    i0 = i0.reshape(IDX_ROWS, 128)
    i1 = i1.reshape(IDX_ROWS, 128)

    energy = functools.partial(
        pl.kernel,
        out_type=jax.ShapeDtypeStruct((NW, L), jnp.float32),
        mesh=mesh,
        compiler_params=pltpu.CompilerParams(needs_layout_passes=False, use_tc_tiling_on_sc=False),
        scratch_types=[
            pltpu.VMEM((BLK_ROWS, 128), jnp.int32),
            pltpu.VMEM((BLK_ROWS, 128), jnp.int32),
            pltpu.VMEM((BLK_PAIRS, ROW_W), jnp.float32),
            pltpu.VMEM((BLK_PAIRS, ROW_W), jnp.float32),
            pltpu.VMEM((BLK_ROWS, 128), jnp.int32),
            pltpu.VMEM((BLK_ROWS, 128), jnp.int32),
            pltpu.VMEM((BLK_PAIRS, ROW_W), jnp.float32),
            pltpu.VMEM((BLK_PAIRS, ROW_W), jnp.float32),
            pltpu.VMEM((L,), jnp.float32),
            pltpu.SemaphoreType.DMA,
            pltpu.SemaphoreType.DMA,
            pltpu.SemaphoreType.DMA,
            pltpu.SemaphoreType.DMA,
        ],
    )(_pair_energy_kernel)
    partials = energy(i0, i1, table)

    return jnp.sum(partials) * (1.0 + weight[0])


# speedup vs baseline: 2.1787x; 1.0559x over previous
import functools
import math

import jax
import jax.numpy as jnp
from jax import lax
from jax.experimental import pallas as pl
from jax.experimental.pallas import tpu as pltpu
from jax.experimental.pallas import tpu_sc as plsc

N_NODES = 100000
N_TYPES = 40
TEMPERATURE = 298.0
ION_STRENGTH = 0.05
CONSTANT = math.exp(-0.004314 * (TEMPERATURE - 273.0))
DIELEC = 8.8
IONIC_CORRECTED = 0.02 + ION_STRENGTH / 1.4
K_SCREEN = math.sqrt(200.0 * IONIC_CORRECTED / TEMPERATURE)
A_COEF = 332.0 / (DIELEC * CONSTANT)

NC = 2
NS = 16
NW = NC * NS
L = 16

ATOMS_PER_TILE = 3136
ATOMS_PAD = NW * ATOMS_PER_TILE

ROW_W = 8
ROWS_PER_TILE = 784
PAIRS_PAD = NW * ROWS_PER_TILE * 128
BLK_ROWS = 16
BLK_PAIRS = BLK_ROWS * 128
N_BLOCKS = ROWS_PER_TILE // BLK_ROWS
IDX_ROWS = PAIRS_PAD // 128


def _widx():
    return lax.axis_index("s") * NC + lax.axis_index("c")


def _iota16():
    return lax.iota(jnp.int32, L)


def _c16(v, dtype=jnp.int32):
    return jnp.full((L,), v, dtype)


def _sqrt16(x):
    i = lax.bitcast_convert_type(x, jnp.int32)
    i = jnp.int32(0x5F3759DF) - lax.shift_right_arithmetic(i, _c16(1))
    y = lax.bitcast_convert_type(i, jnp.float32)
    for _ in range(3):
        y = y * (1.5 - 0.5 * x * y * y)
    return x * y


def _build_table_kernel(coords_hbm, desc_hbm, props_hbm, table_hbm,
                        coords_v, desc_v, props_v, table_v):
    wid = _widx()
    base = wid * ATOMS_PER_TILE
    pltpu.sync_copy(coords_hbm.at[pl.ds(base, ATOMS_PER_TILE)], coords_v)
    pltpu.sync_copy(desc_hbm.at[pl.ds(base, ATOMS_PER_TILE)], desc_v)
    pltpu.sync_copy(props_hbm, props_v)

    def body(j, carry):
        rows = j * L + _iota16()
        atname = plsc.load_gather(desc_v, [rows, _c16(0)])
        resnum = plsc.load_gather(desc_v, [rows, _c16(1)])
        chain = plsc.load_gather(desc_v, [rows, _c16(2)])
        q = plsc.load_gather(props_v, [atname, _c16(1)])
        virt = plsc.load_gather(props_v, [atname, _c16(2)])
        qeff = jnp.where(virt == 1.0, 0.0, q)
        qcode = jnp.where(qeff > 0.0, 2, jnp.where(qeff < 0.0, 0, 1))
        code = qcode + 4 * (resnum * N_TYPES + chain)
        x = plsc.load_gather(coords_v, [rows, _c16(0)])
        y = plsc.load_gather(coords_v, [rows, _c16(1)])
        z = plsc.load_gather(coords_v, [rows, _c16(2)])
        plsc.store_scatter(table_v, [rows, _c16(0)], x)
        plsc.store_scatter(table_v, [rows, _c16(1)], y)
        plsc.store_scatter(table_v, [rows, _c16(2)], z)
        plsc.store_scatter(table_v, [rows, _c16(3)],
                           lax.bitcast_convert_type(code, jnp.float32))
        return carry

    lax.fori_loop(0, ATOMS_PER_TILE // L, body, 0, unroll=False)
    pltpu.sync_copy(table_v, table_hbm.at[pl.ds(base, ATOMS_PER_TILE)])


def _pair_energy_kernel(i0_hbm, i1_hbm, table_hbm, out_hbm,
                        idx0_a, idx1_a, rows0_a, rows1_a,
                        idx0_b, idx1_b, rows0_b, rows1_b,
                        acc_v, sem0_a, sem1_a, sem0_b, sem1_b):
    wid = _widx()
    seta = (idx0_a, idx1_a, rows0_a, rows1_a, sem0_a, sem1_a)
    setb = (idx0_b, idx1_b, rows0_b, rows1_b, sem0_b, sem1_b)

    def issue(bufs, g):
        idx0_v, idx1_v, rows0_v, rows1_v, s0, s1 = bufs
        row_off = wid * ROWS_PER_TILE + g * BLK_ROWS
        pltpu.sync_copy(i0_hbm.at[pl.ds(row_off, BLK_ROWS)], idx0_v)
        pltpu.sync_copy(i1_hbm.at[pl.ds(row_off, BLK_ROWS)], idx1_v)
        for k in range(BLK_ROWS):
            pltpu.async_copy(table_hbm.at[idx0_v.at[k]],
                             rows0_v.at[pl.ds(k * 128, 128)], s0)
            pltpu.async_copy(table_hbm.at[idx1_v.at[k]],
                             rows1_v.at[pl.ds(k * 128, 128)], s1)

    def drain(bufs):
        idx0_v, idx1_v, rows0_v, rows1_v, s0, s1 = bufs
        for k in range(BLK_ROWS):
            pltpu.make_async_copy(table_hbm.at[idx0_v.at[k]],
                                  rows0_v.at[pl.ds(k * 128, 128)], s0).wait()
            pltpu.make_async_copy(table_hbm.at[idx1_v.at[k]],
                                  rows1_v.at[pl.ds(k * 128, 128)], s1).wait()

    def compute(bufs, acc):
        _, _, rows0_v, rows1_v, _, _ = bufs

        def vreg(j, acc):
            rows = j * L + _iota16()
            x0 = plsc.load_gather(rows0_v, [rows, _c16(0)])
            y0 = plsc.load_gather(rows0_v, [rows, _c16(1)])
            z0 = plsc.load_gather(rows0_v, [rows, _c16(2)])
            c0 = lax.bitcast_convert_type(
                plsc.load_gather(rows0_v, [rows, _c16(3)]), jnp.int32)
            x1 = plsc.load_gather(rows1_v, [rows, _c16(0)])
            y1 = plsc.load_gather(rows1_v, [rows, _c16(1)])
            z1 = plsc.load_gather(rows1_v, [rows, _c16(2)])
            c1 = lax.bitcast_convert_type(
                plsc.load_gather(rows1_v, [rows, _c16(3)]), jnp.int32)
            qc0 = c0 & 3
            qc1 = c1 & 3
            meta0 = lax.shift_right_arithmetic(c0, _c16(2))
            meta1 = lax.shift_right_arithmetic(c1, _c16(2))
            dx = x0 - x1
            dy = y0 - y1
            dz = z0 - z1
            d2 = dx * dx + dy * dy + dz * dz
            valid = ((qc0 != 1) & (qc1 != 1) & (meta0 != meta1)
                     & (d2 <= 225.0))
            s = ((qc0 - 1) * (qc1 - 1)).astype(jnp.float32)
            dist = _sqrt16(d2 + 1e-12)
            d28 = jnp.maximum(dist, 2.8)
            e = s * (A_COEF * jnp.exp(-K_SCREEN * d28)) / (d28 * d28)
            return acc + jnp.where(valid, e, 0.0)

        return lax.fori_loop(0, BLK_PAIRS // L, vreg, acc, unroll=False)

    def step(g, acc):
        issue(setb, 2 * g + 1)
        drain(seta)
        acc = compute(seta, acc)
        issue(seta, 2 * g + 2)
        drain(setb)
        return compute(setb, acc)

    issue(seta, 0)
    acc = lax.fori_loop(0, N_BLOCKS // 2, step,
                        jnp.zeros((L,), jnp.float32), unroll=False)
    drain(seta)
    acc = compute(seta, acc)
    acc_v[...] = acc
    pltpu.sync_copy(acc_v, out_hbm.at[wid])


def kernel(coords, partners, atom_description, atomPairs, hbondNet,
           alternativeMask, weight, atom_Properties,
           calculate_helical_dipoles=0):
    del partners, hbondNet, alternativeMask, calculate_helical_dipoles
    mesh = plsc.VectorSubcoreMesh(core_axis_name="c", subcore_axis_name="s")

    coords_p = jnp.zeros((ATOMS_PAD, 3), jnp.float32).at[:N_NODES].set(coords)
    desc_p = jnp.zeros((ATOMS_PAD, 3), jnp.int32).at[:N_NODES].set(
        atom_description)

    build = functools.partial(
        pl.kernel,
        out_type=jax.ShapeDtypeStruct((ATOMS_PAD, ROW_W), jnp.float32),
        mesh=mesh,
        compiler_params=pltpu.CompilerParams(needs_layout_passes=False, use_tc_tiling_on_sc=False),
        scratch_types=[
            pltpu.VMEM((ATOMS_PER_TILE, 3), jnp.float32),
            pltpu.VMEM((ATOMS_PER_TILE, 3), jnp.int32),
            pltpu.VMEM((N_TYPES, 3), jnp.float32),
            pltpu.VMEM((ATOMS_PER_TILE, ROW_W), jnp.float32),
        ],
    )(_build_table_kernel)
    table = build(coords_p, desc_p, atom_Properties)

    n_pairs = atomPairs.shape[0]
    i0 = jnp.zeros((IDX_ROWS * 128,), jnp.int32).at[:n_pairs].set(
        atomPairs[:, 0])
    i1 = jnp.zeros((IDX_ROWS * 128,), jnp.int32).at[:n_pairs].set(
        atomPairs[:, 1])
    i0 = i0.reshape(IDX_ROWS, 128)
    i1 = i1.reshape(IDX_ROWS, 128)

    energy = functools.partial(
        pl.kernel,
        out_type=jax.ShapeDtypeStruct((NW, L), jnp.float32),
        mesh=mesh,
        compiler_params=pltpu.CompilerParams(needs_layout_passes=False, use_tc_tiling_on_sc=False),
        scratch_types=[
            pltpu.VMEM((BLK_ROWS, 128), jnp.int32),
            pltpu.VMEM((BLK_ROWS, 128), jnp.int32),
            pltpu.VMEM((BLK_PAIRS, ROW_W), jnp.float32),
            pltpu.VMEM((BLK_PAIRS, ROW_W), jnp.float32),
            pltpu.VMEM((BLK_ROWS, 128), jnp.int32),
            pltpu.VMEM((BLK_ROWS, 128), jnp.int32),
            pltpu.VMEM((BLK_PAIRS, ROW_W), jnp.float32),
            pltpu.VMEM((BLK_PAIRS, ROW_W), jnp.float32),
            pltpu.VMEM((L,), jnp.float32),
            pltpu.SemaphoreType.DMA,
            pltpu.SemaphoreType.DMA,
            pltpu.SemaphoreType.DMA,
            pltpu.SemaphoreType.DMA,
        ],
    )(_pair_energy_kernel)
    partials = energy(i0, i1, table)

    return jnp.sum(partials) * (1.0 + weight[0])
